# Initial kernel scaffold; baseline (speedup 1.0000x reference)
import jax
import jax.numpy as jnp
from jax.experimental import pallas as pl


def kernel(x, W0, b0, W1, b1, Wout, bout):
    raise NotImplementedError("write your pallas kernel here")


# speedup vs baseline: 6.7644x; 6.7644x over previous
import math

import jax
import jax.numpy as jnp
import numpy as np
from jax.experimental import pallas as pl

_DIM = 128
_MAX_CONN = 4
_OUTPUT_DIM = 64
_TEMPERATURE = 10000
_N_NODES = 32
_N_EDGES = 32
_N_ELT = _N_NODES + _N_EDGES
_BATCH = 4096

_BLK = 256

_edge_order_ids = _N_NODES + np.arange(_N_EDGES)
_ports = np.arange(_N_EDGES) % 4 + 1
_h, _w = _N_ELT, _MAX_CONN + 1
_X, _Y = np.meshgrid(np.arange(_h), np.arange(_w), indexing="xy")
_qc = _DIM // 4
_omega = 1.0 / (_TEMPERATURE ** (np.arange(_qc) / (_qc - 1)))
_y_e = (_Y.flatten()[:, None] * _omega[None, :]).reshape(_h, _w, _qc)
_x_e = (_X.flatten()[:, None] * _omega[None, :]).reshape(_h, _w, _qc)
_EMB = np.concatenate(
    [np.sin(_x_e), np.cos(_x_e), np.sin(_y_e), np.cos(_y_e)], axis=2
).astype(np.float32)
_NPOS = jnp.asarray(_EMB[np.arange(_N_NODES), 0, :])
_EPOS = jnp.asarray(_EMB[_edge_order_ids, _ports, :])


def _body(xe_ref, xo_ref, npos_ref, epos_ref, w0_ref, b0_ref, w1_ref,
          b1_ref, wout_ref, bout_ref, out_ref):
    xe = xe_ref[...]
    xo = xo_ref[...]

    even = (jax.lax.broadcasted_iota(jnp.int32, (1, 1, _DIM), 2) % 2) == 0
    cn = jnp.where(even, xe[:, :_N_NODES, None], xo[:, :_N_NODES, None])
    ce = jnp.where(even, xe[:, _N_NODES:, None], xo[:, _N_NODES:, None])

    h = npos_ref[...][None, :, :] + cn
    e = epos_ref[...][None, :, :] + ce

    for w_ref, b_ref in ((w0_ref, b0_ref), (w1_ref, b1_ref)):
        a = jnp.maximum(h + e, 0.0)
        a_dn = jnp.concatenate([a[:, -1:, :], a[:, :-1, :]], axis=1)
        h_up = jnp.concatenate([h[:, 1:, :], h[:, :1, :]], axis=1)
        b_msg = jnp.maximum(h_up + e, 0.0)
        g = (h + a_dn + b_msg).reshape(_BLK * _N_NODES, _DIM)
        hn = jax.lax.dot_general(
            g, w_ref[...], (((1,), (1,)), ((), ())),
            preferred_element_type=jnp.float32,
            precision=jax.lax.Precision.HIGHEST,
        )
        h = (hn + b_ref[...]).reshape(_BLK, _N_NODES, _DIM)

    pooled = jnp.mean(h, axis=1)
    out = jax.lax.dot_general(
        pooled, wout_ref[...], (((1,), (1,)), ((), ())),
        preferred_element_type=jnp.float32,
        precision=jax.lax.Precision.HIGHEST,
    )
    out_ref[...] = out + bout_ref[...]


def kernel(x, W0, b0, W1, b1, Wout, bout):
    xp = x.reshape(_BATCH, _DIM // 2, 2)
    xe = xp[:, :, 0]
    xo = xp[:, :, 1]

    grid = (_BATCH // _BLK,)
    full = lambda i: (0, 0)
    blk = lambda i: (i, 0)

    return pl.pallas_call(
        _body,
        grid=grid,
        in_specs=[
            pl.BlockSpec((_BLK, _N_ELT), blk),
            pl.BlockSpec((_BLK, _N_ELT), blk),
            pl.BlockSpec((_N_NODES, _DIM), full),
            pl.BlockSpec((_N_EDGES, _DIM), full),
            pl.BlockSpec((_DIM, _DIM), full),
            pl.BlockSpec((1, _DIM), full),
            pl.BlockSpec((_DIM, _DIM), full),
            pl.BlockSpec((1, _DIM), full),
            pl.BlockSpec((_OUTPUT_DIM, _DIM), full),
            pl.BlockSpec((1, _OUTPUT_DIM), full),
        ],
        out_specs=pl.BlockSpec((_BLK, _OUTPUT_DIM), blk),
        out_shape=jax.ShapeDtypeStruct((_BATCH, _OUTPUT_DIM), jnp.float32),
    )(xe, xo, _NPOS, _EPOS, W0, b0.reshape(1, _DIM), W1,
      b1.reshape(1, _DIM), Wout, bout.reshape(1, _OUTPUT_DIM))
